# baseline (speedup 1.0000x reference)
import functools

import jax
import jax.numpy as jnp
from jax import lax
from jax.experimental import pallas as pl
from jax.experimental.pallas import tpu as pltpu
from jax.experimental.pallas import tpu_sc as plsc

NN = 10000
EE = 160000
BB = 256
HH = 256
LL = 4

BLK_N = 1000
BLK_E = 1000
NBLK_N = NN // BLK_N
NBLK_E = EE // BLK_E

_F32 = jnp.float32


def _erf(v):
    return lax.erf(v)


def _gelu(v):
    return 0.5 * v * (1.0 + _erf(v * 0.7071067811865476))


def _proj_body(x_ref, w_ref, b_ref, o_ref, o2_ref):
    r = jnp.dot(x_ref[...], w_ref[...], preferred_element_type=_F32) + b_ref[...]
    o_ref[...] = r
    o2_ref[0] = r[:, :128]
    o2_ref[1] = r[:, 128:]


def _proj(xp, w, b):
    n, k = xp.shape
    h = w.shape[1]
    return pl.pallas_call(
        _proj_body,
        grid=(n // BLK_N,),
        in_specs=[
            pl.BlockSpec((BLK_N, k), lambda i: (i, 0)),
            pl.BlockSpec((k, h), lambda i: (0, 0)),
            pl.BlockSpec((1, h), lambda i: (0, 0)),
        ],
        out_specs=[
            pl.BlockSpec((BLK_N, h), lambda i: (i, 0)),
            pl.BlockSpec((2, BLK_N, 128), lambda i: (0, i, 0)),
        ],
        out_shape=[
            jax.ShapeDtypeStruct((n, h), _F32),
            jax.ShapeDtypeStruct((2, n, 128), _F32),
        ],
    )(xp, w, b)


def _elin_body(ea_ref, bw_ref, bb_ref, lw_ref, lb_ref, o_ref):
    we = jnp.dot(bw_ref[...], lw_ref[...], preferred_element_type=_F32)
    be = jnp.dot(bb_ref[...], lw_ref[...], preferred_element_type=_F32) + lb_ref[...]
    el = jnp.dot(ea_ref[...], we, preferred_element_type=_F32) + be
    o_ref[0] = el[:, :128]
    o_ref[1] = el[:, 128:]


def _elin(eap, bwp, bb, lw, lb):
    k = eap.shape[1]
    return pl.pallas_call(
        _elin_body,
        grid=(NBLK_E,),
        in_specs=[
            pl.BlockSpec((BLK_E, k), lambda i: (i, 0)),
            pl.BlockSpec((k, HH), lambda i: (0, 0)),
            pl.BlockSpec((1, HH), lambda i: (0, 0)),
            pl.BlockSpec((HH, HH), lambda i: (0, 0)),
            pl.BlockSpec((1, HH), lambda i: (0, 0)),
        ],
        out_specs=pl.BlockSpec((2, BLK_E, 128), lambda i: (0, i, 0)),
        out_shape=jax.ShapeDtypeStruct((2, EE, 128), _F32),
    )(eap, bwp, bb, lw, lb)


def _mlp_p1_body(h_ref, a_ref, w1_ref, b1_ref, w2_ref, b2_ref, z_ref, st_ref, acc):
    i = pl.program_id(0)

    @pl.when(i == 0)
    def _():
        acc[...] = jnp.zeros_like(acc)

    z0 = h_ref[...] + jnp.concatenate([a_ref[0], a_ref[1]], axis=1)
    t = _gelu(jnp.dot(z0, w1_ref[...], preferred_element_type=_F32) + b1_ref[...])
    t = jnp.dot(t, w2_ref[...], preferred_element_type=_F32) + b2_ref[...]
    z_ref[...] = t
    acc[0:1, :] += jnp.sum(t, axis=0, keepdims=True)
    acc[1:2, :] += jnp.sum(t * t, axis=0, keepdims=True)

    @pl.when(i == NBLK_N - 1)
    def _():
        st_ref[...] = acc[...]


def _mlp_p1(h, agg, w1, b1, w2, b2):
    return pl.pallas_call(
        _mlp_p1_body,
        grid=(NBLK_N,),
        in_specs=[
            pl.BlockSpec((BLK_N, HH), lambda i: (i, 0)),
            pl.BlockSpec((2, BLK_N, 128), lambda i: (0, i, 0)),
            pl.BlockSpec((HH, 2 * HH), lambda i: (0, 0)),
            pl.BlockSpec((1, 2 * HH), lambda i: (0, 0)),
            pl.BlockSpec((2 * HH, HH), lambda i: (0, 0)),
            pl.BlockSpec((1, HH), lambda i: (0, 0)),
        ],
        out_specs=[
            pl.BlockSpec((BLK_N, HH), lambda i: (i, 0)),
            pl.BlockSpec((2, HH), lambda i: (0, 0)),
        ],
        out_shape=[
            jax.ShapeDtypeStruct((NN, HH), _F32),
            jax.ShapeDtypeStruct((2, HH), _F32),
        ],
        scratch_shapes=[pltpu.VMEM((2, HH), _F32)],
    )(h, agg, w1, b1, w2, b2)


def _mlp_p2_body(z_ref, h_ref, st_ref, gm_ref, bt_ref, o_ref, o2_ref):
    mean = st_ref[0:1, :] * (1.0 / NN)
    var = st_ref[1:2, :] * (1.0 / NN) - mean * mean
    zn = (z_ref[...] - mean) * lax.rsqrt(var + 1e-5) * gm_ref[...] + bt_ref[...]
    r = _gelu(zn) + h_ref[...]
    o_ref[...] = r
    o2_ref[0] = r[:, :128]
    o2_ref[1] = r[:, 128:]


def _mlp_p2(z, h, st, gm, bt):
    return pl.pallas_call(
        _mlp_p2_body,
        grid=(NBLK_N,),
        in_specs=[
            pl.BlockSpec((BLK_N, HH), lambda i: (i, 0)),
            pl.BlockSpec((BLK_N, HH), lambda i: (i, 0)),
            pl.BlockSpec((2, HH), lambda i: (0, 0)),
            pl.BlockSpec((1, HH), lambda i: (0, 0)),
            pl.BlockSpec((1, HH), lambda i: (0, 0)),
        ],
        out_specs=[
            pl.BlockSpec((BLK_N, HH), lambda i: (i, 0)),
            pl.BlockSpec((2, BLK_N, 128), lambda i: (0, i, 0)),
        ],
        out_shape=[
            jax.ShapeDtypeStruct((NN, HH), _F32),
            jax.ShapeDtypeStruct((2, NN, 128), _F32),
        ],
    )(z, h, st, gm, bt)


def _r1_body(h_ref, w1_ref, b1_ref, w2_ref, b2d_ref, s_ref, m_ref, macc):
    i = pl.program_id(0)

    @pl.when(i == 0)
    def _():
        macc[...] = jnp.full_like(macc, -1e30)

    t = jnp.tanh(jnp.dot(h_ref[...], w1_ref[...], preferred_element_type=_F32) + b1_ref[...])
    s_full = jnp.dot(t, w2_ref[...], preferred_element_type=_F32)
    s_ref[...] = s_full
    s_col = s_full[:, :1]
    iota_b = lax.broadcasted_iota(jnp.int32, (1, BB), 1)
    mask = b2d_ref[...] == iota_b
    masked = jnp.where(mask, s_col, -1e30)
    macc[...] = jnp.maximum(macc[...], jnp.max(masked, axis=0, keepdims=True))

    @pl.when(i == NBLK_N - 1)
    def _():
        m_ref[...] = macc[...]


def _readout_p1(h, w1, b1, w2p, batch2d):
    return pl.pallas_call(
        _r1_body,
        grid=(NBLK_N,),
        in_specs=[
            pl.BlockSpec((BLK_N, HH), lambda i: (i, 0)),
            pl.BlockSpec((HH, HH), lambda i: (0, 0)),
            pl.BlockSpec((1, HH), lambda i: (0, 0)),
            pl.BlockSpec((HH, 128), lambda i: (0, 0)),
            pl.BlockSpec((BLK_N, 1), lambda i: (i, 0)),
        ],
        out_specs=[
            pl.BlockSpec((BLK_N, 128), lambda i: (i, 0)),
            pl.BlockSpec((1, BB), lambda i: (0, 0)),
        ],
        out_shape=[
            jax.ShapeDtypeStruct((NN, 128), _F32),
            jax.ShapeDtypeStruct((1, BB), _F32),
        ],
        scratch_shapes=[pltpu.VMEM((1, BB), _F32)],
    )(h, w1, b1, w2p, batch2d)


def _r2_body(h_ref, s_ref, m_ref, b2d_ref, g_ref, gacc, dacc):
    i = pl.program_id(0)

    @pl.when(i == 0)
    def _():
        gacc[...] = jnp.zeros_like(gacc)
        dacc[...] = jnp.zeros_like(dacc)

    iota_b = lax.broadcasted_iota(jnp.int32, (1, BB), 1)
    onehot = (b2d_ref[...] == iota_b).astype(_F32)
    m_sel = jnp.sum(onehot * m_ref[...], axis=1, keepdims=True)
    attn = jnp.exp(s_ref[...][:, :1] - m_sel)
    wh = h_ref[...] * attn
    dn = (((0,), (0,)), ((), ()))
    gacc[...] += lax.dot_general(onehot, wh, dn, preferred_element_type=_F32)
    dacc[...] += lax.dot_general(onehot, attn, dn, preferred_element_type=_F32)

    @pl.when(i == NBLK_N - 1)
    def _():
        d = dacc[...]
        g_ref[...] = jnp.where(d > 0.0, gacc[...] / d, 0.0)


def _readout_p2(h, s_arr, m, batch2d):
    return pl.pallas_call(
        _r2_body,
        grid=(NBLK_N,),
        in_specs=[
            pl.BlockSpec((BLK_N, HH), lambda i: (i, 0)),
            pl.BlockSpec((BLK_N, 128), lambda i: (i, 0)),
            pl.BlockSpec((1, BB), lambda i: (0, 0)),
            pl.BlockSpec((BLK_N, 1), lambda i: (i, 0)),
        ],
        out_specs=pl.BlockSpec((BB, HH), lambda i: (0, 0)),
        out_shape=jax.ShapeDtypeStruct((BB, HH), _F32),
        scratch_shapes=[
            pltpu.VMEM((BB, HH), _F32),
            pltpu.VMEM((BB, 1), _F32),
        ],
    )(h, s_arr, m, batch2d)


_EPT_S = EE // 16
_KS = 40
_NCH = _EPT_S // _KS

_sc_mesh = plsc.VectorSubcoreMesh(core_axis_name="c", subcore_axis_name="s")


@functools.partial(
    pl.kernel,
    out_type=jax.ShapeDtypeStruct((2, NN, 128), _F32),
    mesh=_sc_mesh,
    scratch_types=[
        pltpu.VMEM((_NCH // 2, _KS), jnp.int32),
        pltpu.VMEM((_KS,), jnp.int32),
        pltpu.VMEM((_KS,), jnp.int32),
        pltpu.VMEM((_KS, 128), _F32),
        pltpu.VMEM((_KS, 128), _F32),
        pltpu.VMEM((_KS, 128), _F32),
        pltpu.VMEM((_KS, 128), _F32),
        pltpu.SemaphoreType.DMA,
        pltpu.SemaphoreType.DMA,
        pltpu.SemaphoreType.DMA,
        pltpu.SemaphoreType.DMA,
        pltpu.SemaphoreType.DMA,
        pltpu.SemaphoreType.DMA,
        pltpu.SemaphoreType.DMA,
        pltpu.SemaphoreType.DMA,
        pltpu.VMEM_SHARED((NN, 128), _F32),
    ],
)
def _sc_msgagg(h2_hbm, elin_hbm, src3_hbm, dst_hbm, zeros_hbm, agg_hbm,
               src_v, d0, d1, g0, g1, e0, e1,
               sg0, sg1, se0, se1, sd0, sd1, ss0, ss1, acc):
    c = lax.axis_index("c")
    s = lax.axis_index("s")

    @pl.when(s < 10)
    def _():
        pltpu.sync_copy(
            zeros_hbm.at[pl.ds(s * 1000, 1000)], acc.at[pl.ds(s * 1000, 1000)]
        )

    plsc.subcore_barrier()

    hhalf = h2_hbm.at[c]
    ehalf = elin_hbm.at[c]
    nhalf = _NCH // 2

    def _fire(jloc, base, gbuf, ebuf, dbuf, gsem, esem, dsem):
        cg = pltpu.async_copy(hhalf.at[src_v.at[jloc]], gbuf, gsem)
        ce = pltpu.async_copy(ehalf.at[pl.ds(base, _KS)], ebuf, esem)
        cd = pltpu.async_copy(dst_hbm.at[pl.ds(base, _KS)], dbuf, dsem)
        return cg, ce, cd

    def _relu_add(gbuf, ebuf):
        def rows8(t, carry):
            r0 = t * 8
            for rr in range(8):
                for kk in range(8):
                    sl = pl.ds(kk * 16, 16)
                    ebuf[r0 + rr, sl] = jnp.maximum(
                        ebuf[r0 + rr, sl] + gbuf[r0 + rr, sl], 0.0
                    )
            return carry

        lax.fori_loop(0, _KS // 8, rows8, 0)

    for hhidx in range(2):
        hbase = s * _EPT_S + hhidx * (nhalf * _KS)
        pltpu.sync_copy(src3_hbm.at[s, hhidx], src_v)
        cg0, ce0, cd0 = _fire(0, hbase, g0, e0, d0, sg0, se0, sd0)
        cg1, ce1, cd1 = _fire(1, hbase + _KS, g1, e1, d1, sg1, se1, sd1)

        def steady(t, carry):
            j0 = 2 * t
            j1 = j0 + 1
            pltpu.make_async_copy(
                hhalf.at[src_v.at[j0]], g0, sg0).wait()
            pltpu.make_async_copy(
                ehalf.at[pl.ds(hbase + j0 * _KS, _KS)], e0, se0).wait()
            pltpu.make_async_copy(
                dst_hbm.at[pl.ds(hbase + j0 * _KS, _KS)], d0, sd0).wait()
            _relu_add(g0, e0)
            h0 = pltpu.async_copy(e0, acc.at[d0], ss0, add=True)
            pltpu.make_async_copy(
                hhalf.at[src_v.at[j1]], g1, sg1).wait()
            pltpu.make_async_copy(
                ehalf.at[pl.ds(hbase + j1 * _KS, _KS)], e1, se1).wait()
            pltpu.make_async_copy(
                dst_hbm.at[pl.ds(hbase + j1 * _KS, _KS)], d1, sd1).wait()
            _relu_add(g1, e1)
            h0.wait()

            @pl.when(j0 + 2 < nhalf)
            def _():
                _fire(j0 + 2, hbase + (j0 + 2) * _KS, g0, e0, d0, sg0, se0, sd0)

            h1 = pltpu.async_copy(e1, acc.at[d1], ss1, add=True)
            h1.wait()

            @pl.when(j1 + 2 < nhalf)
            def _():
                _fire(j1 + 2, hbase + (j1 + 2) * _KS, g1, e1, d1, sg1, se1, sd1)

            return carry

        lax.fori_loop(0, nhalf // 2, steady, 0)
        jl = nhalf - 1
        pltpu.make_async_copy(hhalf.at[src_v.at[jl]], g0, sg0).wait()
        pltpu.make_async_copy(
            ehalf.at[pl.ds(hbase + jl * _KS, _KS)], e0, se0).wait()
        pltpu.make_async_copy(
            dst_hbm.at[pl.ds(hbase + jl * _KS, _KS)], d0, sd0).wait()
        _relu_add(g0, e0)
        pltpu.sync_copy(e0, acc.at[d0], add=True)

    plsc.subcore_barrier()

    @pl.when(s < 10)
    def _():
        pltpu.sync_copy(
            acc.at[pl.ds(s * 1000, 1000)], agg_hbm.at[c, pl.ds(s * 1000, 1000)]
        )


def kernel(x, edge_index, edge_attr, batch, atom_W, atom_b, bond_W, bond_b,
           lin_W, lin_b, mlp_W1, mlp_b1, mlp_W2, mlp_b2, bn_gamma, bn_beta,
           gate_W1, gate_b1, gate_W2, gate_b2):
    src = edge_index[0]
    dst = edge_index[1]
    xp = jnp.pad(x, ((0, 0), (0, 9)))
    awp = jnp.pad(atom_W, ((0, 9), (0, 0)))
    eap = jnp.pad(edge_attr, ((0, 0), (0, 2)))
    bwp = jnp.pad(bond_W, ((0, 2), (0, 0)))

    h, h2 = _proj(xp, awp, atom_b.reshape(1, HH))
    zeros_half = jnp.zeros((NN, 128), _F32)
    src3 = src.reshape(16, 2, _NCH // 2, _KS)

    elins = [
        _elin(eap, bwp, bond_b.reshape(1, HH), lin_W[i],
              lin_b[i].reshape(1, HH))
        for i in range(LL)
    ]

    for i in range(LL):
        agg = _sc_msgagg(h2, elins[i], src3, dst, zeros_half)
        z, st = _mlp_p1(h, agg, mlp_W1[i], mlp_b1[i].reshape(1, 2 * HH),
                        mlp_W2[i], mlp_b2[i].reshape(1, HH))
        h, h2 = _mlp_p2(z, h, st, bn_gamma[i].reshape(1, HH),
                        bn_beta[i].reshape(1, HH))

    gw2p = jnp.pad(gate_W2, ((0, 0), (0, 127)))
    batch2d = batch.reshape(NN, 1)
    s_arr, m = _readout_p1(h, gate_W1, gate_b1.reshape(1, HH), gw2p, batch2d)
    g = _readout_p2(h, s_arr, m, batch2d)
    return (h, g)
